# baseline (speedup 1.0000x reference)
import functools

import jax
import jax.numpy as jnp
from jax import lax
from jax.experimental import pallas as pl
from jax.experimental.pallas import tpu as pltpu
from jax.experimental.pallas import tpu_sc as plsc


def _sc_gather_combine(idx1, idx2, entity_embed, relation_embed):
    B = idx1.shape[0]
    D = entity_embed.shape[1]
    H = D // 2
    info = plsc.get_sparse_core_info()
    NC, NS = info.num_cores, info.num_subcores
    NW = NC * NS
    bpw = B // NW

    mesh = plsc.VectorSubcoreMesh(core_axis_name="c", subcore_axis_name="s")

    @functools.partial(
        pl.kernel,
        mesh=mesh,
        compiler_params=pltpu.CompilerParams(use_tc_tiling_on_sc=False),
        out_type=jax.ShapeDtypeStruct((B, D), jnp.float32),
        scratch_types=[
            pltpu.VMEM((bpw,), jnp.int32),
            pltpu.VMEM((bpw,), jnp.int32),
            pltpu.VMEM((bpw, D), jnp.float32),
            pltpu.VMEM((bpw, D), jnp.float32),
            pltpu.VMEM((bpw, D), jnp.float32),
            pltpu.SemaphoreType.DMA,
            pltpu.SemaphoreType.DMA,
        ],
    )
    def body(idx1_hbm, idx2_hbm, ent_hbm, rel_hbm, q_hbm,
             i1_v, i2_v, h_v, r_v, q_v, sem1, sem2):
        wid = lax.axis_index("s") * NC + lax.axis_index("c")
        base = wid * bpw
        pltpu.sync_copy(idx1_hbm.at[pl.ds(base, bpw)], i1_v)
        pltpu.sync_copy(idx2_hbm.at[pl.ds(base, bpw)], i2_v)
        cp_h = pltpu.async_copy(ent_hbm.at[i1_v], h_v, sem1)
        cp_r = pltpu.async_copy(rel_hbm.at[i2_v], r_v, sem2)
        cp_h.wait()
        cp_r.wait()
        for i in range(bpw):
            hr = h_v[i, pl.ds(0, H)]
            hi = h_v[i, pl.ds(H, H)]
            rr = r_v[i, pl.ds(0, H)]
            ri = r_v[i, pl.ds(H, H)]
            q_v[i, pl.ds(0, H)] = hr * rr - hi * ri
            q_v[i, pl.ds(H, H)] = hr * ri + hi * rr
        pltpu.sync_copy(q_v, q_hbm.at[pl.ds(base, bpw)])

    return body(idx1.astype(jnp.int32), idx2.astype(jnp.int32),
                entity_embed, relation_embed)


def _tc_score(q, entity_embed, block_n=2048, nbuf=4):
    B, D = q.shape
    N = entity_embed.shape[0]
    n_full = N // block_n
    tail = N - n_full * block_n
    grid = n_full + (1 if tail else 0)
    tail_slot = n_full % nbuf

    def body(q_ref, e_ref, o_hbm, *scr):
        bufs = scr[:nbuf]
        sems = scr[nbuf:2 * nbuf]
        tail_buf, tail_sem = (scr[2 * nbuf], scr[2 * nbuf + 1]) if tail else (None, None)
        j = pl.program_id(0)
        slot = lax.rem(j, nbuf)

        s = lax.dot_general(q_ref[...], e_ref[...], (((1,), (1,)), ((), ())),
                            preferred_element_type=jnp.float32)
        val = 0.5 * jnp.tanh(0.5 * s) + 0.5

        for i in range(nbuf):
            @pl.when(jnp.logical_and(slot == i, j >= nbuf))
            def _():
                pltpu.make_async_copy(
                    bufs[i], o_hbm.at[:, pl.ds((j - nbuf) * block_n, block_n)],
                    sems[i]).wait()

            @pl.when(jnp.logical_and(slot == i, j < n_full))
            def _():
                bufs[i][...] = val
                pltpu.make_async_copy(
                    bufs[i], o_hbm.at[:, pl.ds(j * block_n, block_n)],
                    sems[i]).start()

        if tail:
            @pl.when(j == n_full)
            def _():
                tail_buf[...] = val[:, :tail]
                pltpu.make_async_copy(
                    tail_buf, o_hbm.at[:, pl.ds(n_full * block_n, tail)],
                    tail_sem).start()

        @pl.when(j == grid - 1)
        def _():
            for i in range(nbuf):
                jj = (n_full - 1) - ((n_full - 1 - i) % nbuf)
                if jj > grid - 1 - nbuf:
                    pltpu.make_async_copy(
                        bufs[i], o_hbm.at[:, pl.ds(jj * block_n, block_n)],
                        sems[i]).wait()
            if tail:
                pltpu.make_async_copy(
                    tail_buf, o_hbm.at[:, pl.ds(n_full * block_n, tail)],
                    tail_sem).wait()

    scratch = ([pltpu.VMEM((B, block_n), jnp.float32) for _ in range(nbuf)]
               + [pltpu.SemaphoreType.DMA for _ in range(nbuf)])
    if tail:
        scratch += [pltpu.VMEM((B, tail), jnp.float32), pltpu.SemaphoreType.DMA]

    return pl.pallas_call(
        body,
        grid=(grid,),
        in_specs=[
            pl.BlockSpec((B, D), lambda j: (0, 0)),
            pl.BlockSpec((block_n, D), lambda j: (j, 0)),
        ],
        out_specs=pl.BlockSpec(memory_space=pl.ANY),
        out_shape=jax.ShapeDtypeStruct((B, N), jnp.float32),
        scratch_shapes=scratch,
    )(q, entity_embed)


def kernel(idx1, idx2, entity_embed, relation_embed):
    q = _sc_gather_combine(idx1, idx2, entity_embed, relation_embed)
    return _tc_score(q, entity_embed)


# speedup vs baseline: 1.0557x; 1.0557x over previous
import functools

import jax
import jax.numpy as jnp
from jax import lax
from jax.experimental import pallas as pl
from jax.experimental.pallas import tpu as pltpu
from jax.experimental.pallas import tpu_sc as plsc


def _sc_gather_combine(idx1, idx2, entity_embed, relation_embed):
    B = idx1.shape[0]
    D = entity_embed.shape[1]
    H = D // 2
    info = plsc.get_sparse_core_info()
    NC, NS = info.num_cores, info.num_subcores
    NW = NC * NS
    bpw = B // NW

    mesh = plsc.VectorSubcoreMesh(core_axis_name="c", subcore_axis_name="s")

    @functools.partial(
        pl.kernel,
        mesh=mesh,
        compiler_params=pltpu.CompilerParams(use_tc_tiling_on_sc=False),
        out_type=jax.ShapeDtypeStruct((B, D), jnp.float32),
        scratch_types=[
            pltpu.VMEM((bpw,), jnp.int32),
            pltpu.VMEM((bpw,), jnp.int32),
            pltpu.VMEM((bpw, D), jnp.float32),
            pltpu.VMEM((bpw, D), jnp.float32),
            pltpu.VMEM((bpw, D), jnp.float32),
            pltpu.SemaphoreType.DMA,
            pltpu.SemaphoreType.DMA,
        ],
    )
    def body(idx1_hbm, idx2_hbm, ent_hbm, rel_hbm, q_hbm,
             i1_v, i2_v, h_v, r_v, q_v, sem1, sem2):
        wid = lax.axis_index("s") * NC + lax.axis_index("c")
        base = wid * bpw
        pltpu.sync_copy(idx1_hbm.at[pl.ds(base, bpw)], i1_v)
        pltpu.sync_copy(idx2_hbm.at[pl.ds(base, bpw)], i2_v)
        cp_h = pltpu.async_copy(ent_hbm.at[i1_v], h_v, sem1)
        cp_r = pltpu.async_copy(rel_hbm.at[i2_v], r_v, sem2)
        cp_h.wait()
        cp_r.wait()
        for i in range(bpw):
            hr = h_v[i, pl.ds(0, H)]
            hi = h_v[i, pl.ds(H, H)]
            rr = r_v[i, pl.ds(0, H)]
            ri = r_v[i, pl.ds(H, H)]
            q_v[i, pl.ds(0, H)] = hr * rr - hi * ri
            q_v[i, pl.ds(H, H)] = hr * ri + hi * rr
        pltpu.sync_copy(q_v, q_hbm.at[pl.ds(base, bpw)])

    return body(idx1.astype(jnp.int32), idx2.astype(jnp.int32),
                entity_embed, relation_embed)


def _tc_score(q, et, block_m=32):
    B, D = q.shape
    N = et.shape[1]

    def body(q_ref, e_ref, o_ref):
        s = lax.dot_general(q_ref[...], e_ref[...], (((1,), (0,)), ((), ())),
                            preferred_element_type=jnp.float32)
        o_ref[...] = 0.5 * jnp.tanh(0.5 * s) + 0.5

    return pl.pallas_call(
        body,
        grid=(B // block_m,),
        in_specs=[
            pl.BlockSpec((block_m, D), lambda i: (i, 0)),
            pl.BlockSpec((D, N), lambda i: (0, 0)),
        ],
        out_specs=pl.BlockSpec((block_m, N), lambda i: (i, 0)),
        out_shape=jax.ShapeDtypeStruct((B, N), jnp.float32),
    )(q, et)


def kernel(idx1, idx2, entity_embed, relation_embed):
    q = _sc_gather_combine(idx1, idx2, entity_embed, relation_embed)
    return _tc_score(q, entity_embed.T)


# speedup vs baseline: 2.5348x; 2.4010x over previous
import functools

import jax
import jax.numpy as jnp
from jax import lax
from jax.experimental import pallas as pl
from jax.experimental.pallas import tpu as pltpu
from jax.experimental.pallas import tpu_sc as plsc


def _sc_gather_combine(idx1, idx2, entity_embed, relation_embed):
    B = idx1.shape[0]
    D = entity_embed.shape[1]
    H = D // 2
    info = plsc.get_sparse_core_info()
    NC, NS = info.num_cores, info.num_subcores
    NW = NC * NS
    bpw = B // NW

    mesh = plsc.VectorSubcoreMesh(core_axis_name="c", subcore_axis_name="s")

    @functools.partial(
        pl.kernel,
        mesh=mesh,
        compiler_params=pltpu.CompilerParams(use_tc_tiling_on_sc=False),
        out_type=jax.ShapeDtypeStruct((B, D), jnp.float32),
        scratch_types=[
            pltpu.VMEM((bpw,), jnp.int32),
            pltpu.VMEM((bpw,), jnp.int32),
            pltpu.VMEM((bpw, D), jnp.float32),
            pltpu.VMEM((bpw, D), jnp.float32),
            pltpu.VMEM((bpw, D), jnp.float32),
            pltpu.SemaphoreType.DMA,
            pltpu.SemaphoreType.DMA,
        ],
    )
    def body(idx1_hbm, idx2_hbm, ent_hbm, rel_hbm, q_hbm,
             i1_v, i2_v, h_v, r_v, q_v, sem1, sem2):
        wid = lax.axis_index("s") * NC + lax.axis_index("c")
        base = wid * bpw
        pltpu.sync_copy(idx1_hbm.at[pl.ds(base, bpw)], i1_v)
        pltpu.sync_copy(idx2_hbm.at[pl.ds(base, bpw)], i2_v)
        cp_h = pltpu.async_copy(ent_hbm.at[i1_v], h_v, sem1)
        cp_r = pltpu.async_copy(rel_hbm.at[i2_v], r_v, sem2)
        cp_h.wait()
        cp_r.wait()
        for i in range(bpw):
            hr = h_v[i, pl.ds(0, H)]
            hi = h_v[i, pl.ds(H, H)]
            rr = r_v[i, pl.ds(0, H)]
            ri = r_v[i, pl.ds(H, H)]
            q_v[i, pl.ds(0, H)] = hr * rr - hi * ri
            q_v[i, pl.ds(H, H)] = hr * ri + hi * rr
        pltpu.sync_copy(q_v, q_hbm.at[pl.ds(base, bpw)])

    return body(idx1.astype(jnp.int32), idx2.astype(jnp.int32),
                entity_embed, relation_embed)


def _tc_score(q, entity_embed, block_n=2048):
    B, D = q.shape
    N = entity_embed.shape[0]

    def body(e_ref, q_ref, o_ref):
        s = lax.dot_general(e_ref[...], q_ref[...], (((1,), (1,)), ((), ())),
                            preferred_element_type=jnp.float32)
        o_ref[...] = 0.5 * jnp.tanh(0.5 * s) + 0.5

    pT = pl.pallas_call(
        body,
        grid=(pl.cdiv(N, block_n),),
        in_specs=[
            pl.BlockSpec((block_n, D), lambda i: (i, 0)),
            pl.BlockSpec((B, D), lambda i: (0, 0)),
        ],
        out_specs=pl.BlockSpec((block_n, B), lambda i: (i, 0)),
        out_shape=jax.ShapeDtypeStruct((N, B), jnp.float32),
    )(entity_embed, q)
    return pT.T


def kernel(idx1, idx2, entity_embed, relation_embed):
    q = _sc_gather_combine(idx1, idx2, entity_embed, relation_embed)
    return _tc_score(q, entity_embed)


# speedup vs baseline: 2.5360x; 1.0005x over previous
import functools

import jax
import jax.numpy as jnp
from jax import lax
from jax.experimental import pallas as pl
from jax.experimental.pallas import tpu as pltpu
from jax.experimental.pallas import tpu_sc as plsc


def _sc_gather_combine(idx1, idx2, entity_embed, relation_embed):
    B = idx1.shape[0]
    D = entity_embed.shape[1]
    H = D // 2
    info = plsc.get_sparse_core_info()
    NC, NS = info.num_cores, info.num_subcores
    NW = NC * NS
    bpw = B // NW

    mesh = plsc.VectorSubcoreMesh(core_axis_name="c", subcore_axis_name="s")

    @functools.partial(
        pl.kernel,
        mesh=mesh,
        compiler_params=pltpu.CompilerParams(use_tc_tiling_on_sc=False),
        out_type=jax.ShapeDtypeStruct((B, D), jnp.float32),
        scratch_types=[
            pltpu.VMEM((bpw,), jnp.int32),
            pltpu.VMEM((bpw,), jnp.int32),
            pltpu.VMEM((bpw, D), jnp.float32),
            pltpu.VMEM((bpw, D), jnp.float32),
            pltpu.VMEM((bpw, D), jnp.float32),
            pltpu.SemaphoreType.DMA,
            pltpu.SemaphoreType.DMA,
        ],
    )
    def body(idx1_hbm, idx2_hbm, ent_hbm, rel_hbm, q_hbm,
             i1_v, i2_v, h_v, r_v, q_v, sem1, sem2):
        wid = lax.axis_index("s") * NC + lax.axis_index("c")
        base = wid * bpw
        pltpu.sync_copy(idx1_hbm.at[pl.ds(base, bpw)], i1_v)
        pltpu.sync_copy(idx2_hbm.at[pl.ds(base, bpw)], i2_v)
        cp_h = pltpu.async_copy(ent_hbm.at[i1_v], h_v, sem1)
        cp_r = pltpu.async_copy(rel_hbm.at[i2_v], r_v, sem2)
        cp_h.wait()
        cp_r.wait()
        for i in range(bpw):
            hr = h_v[i, pl.ds(0, H)]
            hi = h_v[i, pl.ds(H, H)]
            rr = r_v[i, pl.ds(0, H)]
            ri = r_v[i, pl.ds(H, H)]
            q_v[i, pl.ds(0, H)] = hr * rr - hi * ri
            q_v[i, pl.ds(H, H)] = hr * ri + hi * rr
        pltpu.sync_copy(q_v, q_hbm.at[pl.ds(base, bpw)])

    return body(idx1.astype(jnp.int32), idx2.astype(jnp.int32),
                entity_embed, relation_embed)


def _tc_score(q, entity_embed, block_n=2048):
    B, D = q.shape
    N = entity_embed.shape[0]

    def body(e_ref, q_ref, o_ref):
        s = lax.dot_general(e_ref[...], q_ref[...], (((1,), (1,)), ((), ())),
                            preferred_element_type=jnp.float32)
        o_ref[...] = 0.5 * jnp.tanh(0.5 * s) + 0.5

    pT = pl.pallas_call(
        body,
        grid=(pl.cdiv(N, block_n),),
        in_specs=[
            pl.BlockSpec((block_n, D), lambda i: (i, 0)),
            pl.BlockSpec((B, D), lambda i: (0, 0)),
        ],
        out_specs=pl.BlockSpec((block_n, B), lambda i: (i, 0)),
        out_shape=jax.ShapeDtypeStruct((N, B), jnp.float32),
    )(entity_embed, q)
    return pT.T


def kernel(idx1, idx2, entity_embed, relation_embed):
    e4 = lax.optimization_barrier(entity_embed.reshape(25000, 128))
    ent_sc = e4.reshape(100000, 32)
    q = _sc_gather_combine(idx1, idx2, ent_sc, relation_embed)
    return _tc_score(q, entity_embed)


# speedup vs baseline: 2.6619x; 1.0496x over previous
import functools

import jax
import jax.numpy as jnp
from jax import lax
from jax.experimental import pallas as pl
from jax.experimental.pallas import tpu as pltpu
from jax.experimental.pallas import tpu_sc as plsc


def _sc_gather_combine(idx1, idx2, entity_embed, relation_embed):
    B = idx1.shape[0]
    D = entity_embed.shape[1]
    H = D // 2
    info = plsc.get_sparse_core_info()
    NC, NS = info.num_cores, info.num_subcores
    NW = NC * NS
    bpw = B // NW

    mesh = plsc.VectorSubcoreMesh(core_axis_name="c", subcore_axis_name="s")

    @functools.partial(
        pl.kernel,
        mesh=mesh,
        compiler_params=pltpu.CompilerParams(use_tc_tiling_on_sc=False),
        out_type=jax.ShapeDtypeStruct((B, D), jnp.float32),
        scratch_types=[
            pltpu.VMEM((bpw,), jnp.int32),
            pltpu.VMEM((bpw,), jnp.int32),
            pltpu.VMEM((bpw, D), jnp.float32),
            pltpu.VMEM((bpw, D), jnp.float32),
            pltpu.VMEM((bpw, D), jnp.float32),
            pltpu.SemaphoreType.DMA,
            pltpu.SemaphoreType.DMA,
        ],
    )
    def body(idx1_hbm, idx2_hbm, ent_hbm, rel_hbm, q_hbm,
             i1_v, i2_v, h_v, r_v, q_v, sem1, sem2):
        wid = lax.axis_index("s") * NC + lax.axis_index("c")
        base = wid * bpw
        pltpu.sync_copy(idx1_hbm.at[pl.ds(base, bpw)], i1_v)
        pltpu.sync_copy(idx2_hbm.at[pl.ds(base, bpw)], i2_v)
        cp_h = pltpu.async_copy(ent_hbm.at[i1_v], h_v, sem1)
        cp_r = pltpu.async_copy(rel_hbm.at[i2_v], r_v, sem2)
        cp_h.wait()
        cp_r.wait()
        for i in range(bpw):
            hr = h_v[i, pl.ds(0, H)]
            hi = h_v[i, pl.ds(H, H)]
            rr = r_v[i, pl.ds(0, H)]
            ri = r_v[i, pl.ds(H, H)]
            q_v[i, pl.ds(0, H)] = hr * rr - hi * ri
            q_v[i, pl.ds(H, H)] = hr * ri + hi * rr
        pltpu.sync_copy(q_v, q_hbm.at[pl.ds(base, bpw)])

    return body(idx1.astype(jnp.int32), idx2.astype(jnp.int32),
                entity_embed, relation_embed)


def _tc_score(q, entity_embed, block_n=2048):
    B, D = q.shape
    N = entity_embed.shape[0]

    def body(e_ref, q_ref, o_ref):
        s = lax.dot_general(e_ref[...], q_ref[...], (((1,), (1,)), ((), ())),
                            preferred_element_type=jnp.float32)
        o_ref[...] = 0.5 * jnp.tanh(0.5 * s) + 0.5

    pT = pl.pallas_call(
        body,
        grid=(pl.cdiv(N, block_n),),
        in_specs=[
            pl.BlockSpec((block_n, D), lambda i: (i, 0)),
            pl.BlockSpec((B, D), lambda i: (0, 0)),
        ],
        out_specs=pl.BlockSpec((block_n, B), lambda i: (i, 0)),
        out_shape=jax.ShapeDtypeStruct((N, B), jnp.float32),
    )(entity_embed, q)
    return pT.T


def _tc_pack4(entity_embed, block_n=8192):
    N, D = entity_embed.shape

    def body(e_ref, o_ref):
        o_ref[...] = jnp.concatenate(
            [e_ref[pl.Slice(j, block_n // 4, 4), :] for j in range(4)], axis=1)

    return pl.pallas_call(
        body,
        grid=(N // block_n,),
        in_specs=[pl.BlockSpec((block_n, D), lambda i: (i, 0))],
        out_specs=pl.BlockSpec((block_n // 4, 4 * D), lambda i: (i, 0)),
        out_shape=jax.ShapeDtypeStruct((N // 4, 4 * D), jnp.float32),
    )(entity_embed)


def kernel(idx1, idx2, entity_embed, relation_embed):
    e4 = _tc_pack4(entity_embed)
    ent_sc = e4.reshape(100000, 32)
    q = _sc_gather_combine(idx1, idx2, ent_sc, relation_embed)
    return _tc_score(q, entity_embed)
